# initial kernel scaffold (unmeasured)
import jax
import jax.numpy as jnp
from jax import lax
from jax.experimental import pallas as pl
from jax.experimental.pallas import tpu as pltpu


def kernel(
    x,
):
    def body(*refs):
        pass

    out_shape = jax.ShapeDtypeStruct(..., jnp.float32)
    return pl.pallas_call(body, out_shape=out_shape)(...)


# baseline (device time: 32262 ns/iter reference)
import jax
import jax.numpy as jnp
from jax import lax
from jax.experimental import pallas as pl
from jax.experimental.pallas import tpu as pltpu

N_DEV = 16


def kernel(x):
    m_per, n = x.shape

    def body(x_ref, out_ref, comm_ref, send_sems, recv_sems):
        my = lax.axis_index("i")
        left = (my - 1) % N_DEV
        right = (my + 1) % N_DEV

        barrier_sem = pltpu.get_barrier_semaphore()
        for nbr in [left, right]:
            pl.semaphore_signal(
                barrier_sem, inc=1,
                device_id=(nbr,), device_id_type=pl.DeviceIdType.MESH,
            )
        pl.semaphore_wait(barrier_sem, 2)

        xv = x_ref[:, :]
        val = jnp.max(xv, axis=0)
        row_ids = lax.broadcasted_iota(jnp.int32, (m_per, n), 0)
        loc_idx = jnp.min(
            jnp.where(xv == val[None, :], row_ids, m_per), axis=0
        )
        gidx = (loc_idx + my * m_per).astype(jnp.float32)

        best_val = val
        best_idx = gidx
        comm_ref[0, 0, :] = val
        comm_ref[0, 1, :] = gidx

        for h in range(N_DEV - 1):
            send_slot = h % 2
            recv_slot = (h + 1) % 2
            rdma = pltpu.make_async_remote_copy(
                src_ref=comm_ref.at[send_slot],
                dst_ref=comm_ref.at[recv_slot],
                send_sem=send_sems.at[send_slot],
                recv_sem=recv_sems.at[recv_slot],
                device_id=(right,),
                device_id_type=pl.DeviceIdType.MESH,
            )
            rdma.start()
            rdma.wait()

            inc_val = comm_ref[recv_slot, 0, :]
            inc_idx = comm_ref[recv_slot, 1, :]
            take = inc_val > best_val
            best_val = jnp.where(take, inc_val, best_val)
            best_idx = jnp.where(take, inc_idx, best_idx)

        out_ref[0, :] = best_val
        out_ref[1, :] = best_idx

    return pl.pallas_call(
        body,
        out_shape=jax.ShapeDtypeStruct((2, n), jnp.float32),
        in_specs=[pl.BlockSpec(memory_space=pltpu.VMEM)],
        out_specs=pl.BlockSpec(memory_space=pltpu.VMEM),
        scratch_shapes=[
            pltpu.VMEM((2, 2, n), jnp.float32),
            pltpu.SemaphoreType.DMA((2,)),
            pltpu.SemaphoreType.DMA((2,)),
        ],
        compiler_params=pltpu.CompilerParams(collective_id=0),
    )(x)


# device time: 13872 ns/iter; 2.3257x vs baseline; 2.3257x over previous
import jax
import jax.numpy as jnp
from jax import lax
from jax.experimental import pallas as pl
from jax.experimental.pallas import tpu as pltpu

N_DEV = 16
N_ROUNDS = 4


def kernel(x):
    m_per, n = x.shape

    def body(x_ref, out_ref, send_ref, recv_ref, send_sems, recv_sems):
        my = lax.axis_index("i")

        barrier_sem = pltpu.get_barrier_semaphore()
        for r in range(N_ROUNDS):
            partner = jnp.bitwise_xor(my, 1 << r)
            pl.semaphore_signal(
                barrier_sem, inc=1,
                device_id=(partner,), device_id_type=pl.DeviceIdType.MESH,
            )
        pl.semaphore_wait(barrier_sem, N_ROUNDS)

        xv = x_ref[:, :]
        val = jnp.max(xv, axis=0)
        row_ids = lax.broadcasted_iota(jnp.int32, (m_per, n), 0)
        loc_idx = jnp.min(
            jnp.where(xv == val[None, :], row_ids, m_per), axis=0
        )
        best_val = val
        best_idx = (loc_idx + my * m_per).astype(jnp.float32)

        for r in range(N_ROUNDS):
            partner = jnp.bitwise_xor(my, 1 << r)
            send_ref[r, 0, :] = best_val
            send_ref[r, 1, :] = best_idx
            rdma = pltpu.make_async_remote_copy(
                src_ref=send_ref.at[r],
                dst_ref=recv_ref.at[r],
                send_sem=send_sems.at[r],
                recv_sem=recv_sems.at[r],
                device_id=(partner,),
                device_id_type=pl.DeviceIdType.MESH,
            )
            rdma.start()
            rdma.wait_recv()

            inc_val = recv_ref[r, 0, :]
            inc_idx = recv_ref[r, 1, :]
            take = inc_val > best_val
            best_val = jnp.where(take, inc_val, best_val)
            best_idx = jnp.where(take, inc_idx, best_idx)
            rdma.wait_send()

        out_ref[0, :] = best_val
        out_ref[1, :] = best_idx

    return pl.pallas_call(
        body,
        out_shape=jax.ShapeDtypeStruct((2, n), jnp.float32),
        in_specs=[pl.BlockSpec(memory_space=pltpu.VMEM)],
        out_specs=pl.BlockSpec(memory_space=pltpu.VMEM),
        scratch_shapes=[
            pltpu.VMEM((N_ROUNDS, 2, n), jnp.float32),
            pltpu.VMEM((N_ROUNDS, 2, n), jnp.float32),
            pltpu.SemaphoreType.DMA((N_ROUNDS,)),
            pltpu.SemaphoreType.DMA((N_ROUNDS,)),
        ],
        compiler_params=pltpu.CompilerParams(collective_id=0),
    )(x)


# device time: 9673 ns/iter; 3.3353x vs baseline; 1.4341x over previous
import jax
import jax.numpy as jnp
from jax import lax
from jax.experimental import pallas as pl
from jax.experimental.pallas import tpu as pltpu

N_DEV = 16


def kernel(x):
    m_per, n = x.shape

    def body(x_ref, out_ref, send_ref, recv_ref, send_sems, recv_sems):
        my = lax.axis_index("i")

        barrier_sem = pltpu.get_barrier_semaphore()
        for e in range(1, N_DEV):
            pl.semaphore_signal(
                barrier_sem, inc=1,
                device_id=((my + e) % N_DEV,),
                device_id_type=pl.DeviceIdType.MESH,
            )

        xv = x_ref[:, :]
        val = jnp.max(xv, axis=0)
        row_ids = lax.broadcasted_iota(jnp.int32, (m_per, n), 0)
        loc_idx = jnp.min(
            jnp.where(xv == val[None, :], row_ids, m_per), axis=0
        )
        best_val = val
        best_idx = (loc_idx + my * m_per).astype(jnp.float32)
        send_ref[0, :] = best_val
        send_ref[1, :] = best_idx

        pl.semaphore_wait(barrier_sem, N_DEV - 1)

        rdmas = []
        for e in range(1, N_DEV):
            rdma = pltpu.make_async_remote_copy(
                src_ref=send_ref,
                dst_ref=recv_ref.at[e],
                send_sem=send_sems.at[e],
                recv_sem=recv_sems.at[e],
                device_id=((my + e) % N_DEV,),
                device_id_type=pl.DeviceIdType.MESH,
            )
            rdma.start()
            rdmas.append(rdma)

        for e in range(1, N_DEV):
            rdmas[e - 1].wait_recv()
            inc_val = recv_ref[e, 0, :]
            inc_idx = recv_ref[e, 1, :]
            take = inc_val > best_val
            best_val = jnp.where(take, inc_val, best_val)
            best_idx = jnp.where(take, inc_idx, best_idx)

        out_ref[0, :] = best_val
        out_ref[1, :] = best_idx

        for r in rdmas:
            r.wait_send()

    return pl.pallas_call(
        body,
        out_shape=jax.ShapeDtypeStruct((2, n), jnp.float32),
        in_specs=[pl.BlockSpec(memory_space=pltpu.VMEM)],
        out_specs=pl.BlockSpec(memory_space=pltpu.VMEM),
        scratch_shapes=[
            pltpu.VMEM((2, n), jnp.float32),
            pltpu.VMEM((N_DEV, 2, n), jnp.float32),
            pltpu.SemaphoreType.DMA((N_DEV,)),
            pltpu.SemaphoreType.DMA((N_DEV,)),
        ],
        compiler_params=pltpu.CompilerParams(collective_id=0),
    )(x)
